# Initial kernel scaffold; baseline (speedup 1.0000x reference)
import jax
import jax.numpy as jnp
from jax.experimental import pallas as pl


def kernel(input, embeddings):
    raise NotImplementedError("write your pallas kernel here")


# speedup vs baseline: 1.0942x; 1.0942x over previous
import functools

import jax
import jax.numpy as jnp
from jax import lax
from jax.experimental import pallas as pl
from jax.experimental.pallas import tpu as pltpu
from jax.experimental.pallas import tpu_sc as plsc

NC = 2
NS = 16
NW = NC * NS

IDX_ROW = 128
CHUNK_ROWS = 8
CHUNK = IDX_ROW * CHUNK_ROWS


def _gather_body(table_hbm, idx_hbm, out_hbm, idx_v, rows_v, sem):
    D = table_hbm.shape[1]
    n_rows = idx_hbm.shape[0]
    rows_per_w = n_rows // NW
    chunks = rows_per_w // CHUNK_ROWS

    wid = lax.axis_index("s") * NC + lax.axis_index("c")
    row0 = wid * rows_per_w

    def chunk_step(c, carry):
        rbase = row0 + c * CHUNK_ROWS
        pltpu.sync_copy(idx_hbm.at[pl.ds(rbase, CHUNK_ROWS)], idx_v)
        copies = []
        for j in range(CHUNK_ROWS):
            copies.append(
                pltpu.async_copy(
                    table_hbm.at[idx_v.at[j]],
                    rows_v.at[pl.ds(j * IDX_ROW, IDX_ROW)],
                    sem,
                )
            )
        for cp in copies:
            cp.wait()
        pltpu.sync_copy(rows_v, out_hbm.at[pl.ds(rbase * IDX_ROW, CHUNK)])
        return carry

    lax.fori_loop(0, chunks, chunk_step, 0)


@functools.partial(jax.jit, static_argnames=())
def kernel(input, embeddings):
    orig_shape = input.shape
    D = embeddings.shape[1]
    flat_idx = input.reshape(-1).astype(jnp.int32)
    B = flat_idx.shape[0]
    idx2d = flat_idx.reshape(B // IDX_ROW, IDX_ROW)

    mesh = plsc.VectorSubcoreMesh(
        core_axis_name="c", subcore_axis_name="s",
        num_cores=NC, num_subcores=NS,
    )
    out = pl.kernel(
        _gather_body,
        out_type=jax.ShapeDtypeStruct((B, D), jnp.float32),
        mesh=mesh,
        scratch_types=[
            pltpu.VMEM((CHUNK_ROWS, IDX_ROW), jnp.int32),
            pltpu.VMEM((CHUNK, D), jnp.float32),
            pltpu.SemaphoreType.DMA,
        ],
        compiler_params=pltpu.CompilerParams(use_tc_tiling_on_sc=False),
    )(embeddings, idx2d)
    return out.reshape(*orig_shape, D)


# speedup vs baseline: 1.1107x; 1.0151x over previous
import functools

import jax
import jax.numpy as jnp
from jax import lax
from jax.experimental import pallas as pl
from jax.experimental.pallas import tpu as pltpu
from jax.experimental.pallas import tpu_sc as plsc

NC = 2
NS = 16
NW = NC * NS

IDX_ROW = 128
CHUNK_ROWS = 10
CHUNK = IDX_ROW * CHUNK_ROWS
NBUF = 2


def _gather_body(table_hbm, idx_hbm, out_hbm,
                 idx_v, rows_v, isem0, isem1, gsem0, gsem1, osem0, osem1):
    D = table_hbm.shape[1]
    n_rows = idx_hbm.shape[0]
    rows_per_w = n_rows // NW
    chunks = rows_per_w // CHUNK_ROWS
    G = chunks // NBUF

    isems = (isem0, isem1)
    gsems = (gsem0, gsem1)
    osems = (osem0, osem1)

    wid = lax.axis_index("s") * NC + lax.axis_index("c")
    row0 = wid * rows_per_w

    def idx_start(c, b):
        pltpu.async_copy(
            idx_hbm.at[pl.ds(row0 + c * CHUNK_ROWS, CHUNK_ROWS)],
            idx_v.at[b], isems[b])

    def chunk_work(c, b, wait_out, prefetch):
        idx_buf = idx_v.at[b]
        rows_buf = rows_v.at[b]
        pltpu.make_async_copy(
            idx_hbm.at[pl.ds(0, CHUNK_ROWS)], idx_buf, isems[b]).wait()
        if wait_out:
            pltpu.make_async_copy(
                rows_buf, out_hbm.at[pl.ds(0, CHUNK)], osems[b]).wait()
        copies = []
        for j in range(CHUNK_ROWS):
            copies.append(pltpu.async_copy(
                table_hbm.at[idx_buf.at[j]],
                rows_buf.at[pl.ds(j * IDX_ROW, IDX_ROW)], gsems[b]))
        for cp in copies:
            cp.wait()
        if prefetch:
            idx_start(c + NBUF, b)
        pltpu.async_copy(
            rows_buf,
            out_hbm.at[pl.ds((row0 + c * CHUNK_ROWS) * IDX_ROW, CHUNK)],
            osems[b])

    idx_start(0, 0)
    idx_start(1, 1)
    chunk_work(0, 0, wait_out=False, prefetch=True)
    chunk_work(1, 1, wait_out=False, prefetch=True)

    def step(g, carry):
        for b in range(NBUF):
            chunk_work(g * NBUF + b, b, wait_out=True, prefetch=True)
        return carry

    lax.fori_loop(1, G - 1, step, 0)

    chunk_work((G - 1) * NBUF, 0, wait_out=True, prefetch=False)
    chunk_work((G - 1) * NBUF + 1, 1, wait_out=True, prefetch=False)
    for b in range(NBUF):
        pltpu.make_async_copy(
            rows_v.at[b], out_hbm.at[pl.ds(0, CHUNK)], osems[b]).wait()


@functools.partial(jax.jit, static_argnames=())
def kernel(input, embeddings):
    orig_shape = input.shape
    D = embeddings.shape[1]
    flat_idx = input.reshape(-1).astype(jnp.int32)
    B = flat_idx.shape[0]
    idx2d = flat_idx.reshape(B // IDX_ROW, IDX_ROW)

    mesh = plsc.VectorSubcoreMesh(
        core_axis_name="c", subcore_axis_name="s",
        num_cores=NC, num_subcores=NS,
    )
    out = pl.kernel(
        _gather_body,
        out_type=jax.ShapeDtypeStruct((B, D), jnp.float32),
        mesh=mesh,
        scratch_types=[
            pltpu.VMEM((NBUF, CHUNK_ROWS, IDX_ROW), jnp.int32),
            pltpu.VMEM((NBUF, CHUNK, D), jnp.float32),
            pltpu.SemaphoreType.DMA,
            pltpu.SemaphoreType.DMA,
            pltpu.SemaphoreType.DMA,
            pltpu.SemaphoreType.DMA,
            pltpu.SemaphoreType.DMA,
            pltpu.SemaphoreType.DMA,
        ],
        compiler_params=pltpu.CompilerParams(use_tc_tiling_on_sc=False),
    )(embeddings, idx2d)
    return out.reshape(*orig_shape, D)
